# baseline (speedup 1.0000x reference)
import functools

import jax
import jax.numpy as jnp
from jax import lax
from jax.experimental import pallas as pl
from jax.experimental.pallas import tpu as pltpu
from jax.experimental.pallas import tpu_sc as plsc

_N = 1000
_NP = 1024
_K_OUT = 100
_THRESH = 0.3
_MASK_D = 28 * 28 * 28


def _detect_body(s_row_ref, s_col_ref, props_ref, deltas_ref, props_t_ref,
                 deltas_t_ref, boxes_out_ref, gidx_out_ref, iou_ref):
    f32 = jnp.float32
    s_row = s_row_ref[0]
    s_col = s_col_ref[0]
    lane = lax.broadcasted_iota(jnp.int32, (_NP, _NP), 1)
    sub = lax.broadcasted_iota(jnp.int32, (_NP, _NP), 0)

    cmp = jnp.where((s_row > s_col) | ((s_row == s_col) & (lane < sub)),
                    f32(1.0), f32(0.0))
    rank_col = jnp.sum(cmp, axis=1, keepdims=True)
    cmp_t = jnp.where((s_col > s_row) | ((s_col == s_row) & (sub < lane)),
                      f32(1.0), f32(0.0))
    rank_row = jnp.sum(cmp_t, axis=0, keepdims=True)

    sub_f = sub.astype(f32)
    lane_f = lane.astype(f32)
    perm = jnp.where(rank_row == sub_f, f32(1.0), f32(0.0))
    perm_t = jnp.where(rank_col == lane_f, f32(1.0), f32(0.0))

    dot = functools.partial(jnp.dot, preferred_element_type=f32,
                            precision=lax.Precision.HIGHEST)

    def regress(p, d, axis):
        def g(a, i):
            if axis == 1:
                return lax.slice_in_dim(a, i, i + 1, axis=1)
            return lax.slice_in_dim(a, i, i + 1, axis=0)
        y1, x1, z1, y2, x2, z2 = (g(p, i) for i in range(6))
        dy, dx, dz, dh, dw, dd = (g(d, i) for i in range(6))
        h = y2 - y1
        w = x2 - x1
        dep = z2 - z1
        cy = y1 + 0.5 * h + dy * h
        cx = x1 + 0.5 * w + dx * w
        cz = z1 + 0.5 * dep + dz * dep
        h = h * jnp.exp(dh)
        w = w * jnp.exp(dw)
        dep = dep * jnp.exp(dd)
        parts = [cy - 0.5 * h, cx - 0.5 * w, cz - 0.5 * dep,
                 cy + 0.5 * h, cx + 0.5 * w, cz + 0.5 * dep]
        return jnp.concatenate(parts, axis=1 if axis == 1 else 0)

    bb_col = regress(props_ref[0], deltas_ref[0], axis=1)
    bb_row = regress(props_t_ref[0], deltas_t_ref[0], axis=0)
    sbox_c = dot(perm, bb_col)
    sbox_r = dot(bb_row, perm_t)

    def col(i):
        return lax.slice_in_dim(sbox_c, i, i + 1, axis=1)

    def row(i):
        return lax.slice_in_dim(sbox_r, i, i + 1, axis=0)

    y1c, x1c, z1c, y2c, x2c, z2c = (col(i) for i in range(6))
    y1r, x1r, z1r, y2r, x2r, z2r = (row(i) for i in range(6))
    zero = f32(0.0)
    inter = (jnp.maximum(jnp.minimum(y2c, y2r) - jnp.maximum(y1c, y1r), zero)
             * jnp.maximum(jnp.minimum(x2c, x2r) - jnp.maximum(x1c, x1r), zero)
             * jnp.maximum(jnp.minimum(z2c, z2r) - jnp.maximum(z1c, z1r), zero))
    vol_c = (y2c - y1c) * (x2c - x1c) * (z2c - z1c)
    vol_r = (y2r - y1r) * (x2r - x1r) * (z2r - z1r)
    iou_ref[...] = inter / (vol_c + vol_r - inter + f32(1e-8))

    lane_row = lax.broadcasted_iota(jnp.int32, (1, _NP), 1)
    lane_row_f = lane_row.astype(f32)

    def nms_step(i, keep):
        iou_row = iou_ref[pl.ds(i, 1), :]
        keep_i = jnp.sum(jnp.where(lane_row == i, keep, zero))
        suppress = (iou_row > _THRESH) & (lane_row > i) & (keep_i > zero)
        return jnp.where(suppress, zero, keep)

    keep = lax.fori_loop(0, _N, nms_step, jnp.ones((1, _NP), f32))

    validm = lane_row < _N
    keptf = jnp.where(validm, keep, zero)
    suppf = jnp.where(validm, 1.0 - keep, zero)
    tri = jnp.where(sub <= lane, f32(1.0), f32(0.0))
    csum_kept = dot(keptf, tri)
    csum_supp = dot(suppf, tri)
    n_kept = jnp.sum(keptf)
    slot = jnp.where(keptf > zero, csum_kept - 1.0,
                     jnp.where(suppf > zero, n_kept + csum_supp - 1.0,
                               f32(1e9)))

    sel_sub = lax.broadcasted_iota(jnp.int32, (128, _NP), 0).astype(f32)
    sel = jnp.where(slot == sel_sub, f32(1.0), f32(0.0))
    boxes_out_ref[0] = dot(sel, sbox_c)
    sorted_gidx = dot(perm, sub_f[:, 0:1])
    gidx_out_ref[0] = dot(sel, sorted_gidx)


_SC_NC = 2
_SC_NS = 16
_SC_NW = _SC_NC * _SC_NS
_MASK_DP = 22016
_SUB_K = 4
_SUB_D = _MASK_DP // _SUB_K
_SUB_TOT = 1024
_SUB_W = _SUB_TOT // _SC_NW
_SUB_CH = 8


def _sc_gather_body(table_hbm, idx_hbm, out_hbm, idx_v, buf0, buf1, sem0, sem1):
    wid = lax.axis_index("s") * _SC_NC + lax.axis_index("c")
    base = wid * _SUB_W
    pltpu.sync_copy(idx_hbm.at[pl.ds(base, _SUB_W)], idx_v)
    bufs = (buf0, buf1)
    sems = (sem0, sem1)
    n_ch = _SUB_W // _SUB_CH
    handles = [None] * n_ch

    def start(c):
        handles[c] = pltpu.make_async_copy(
            table_hbm.at[idx_v.at[pl.ds(c * _SUB_CH, _SUB_CH)]],
            bufs[c % 2], sems[c % 2])
        handles[c].start()

    start(0)
    if n_ch > 1:
        start(1)
    for c in range(n_ch):
        handles[c].wait()
        pltpu.sync_copy(bufs[c % 2],
                        out_hbm.at[pl.ds(base + c * _SUB_CH, _SUB_CH)])
        if c + 2 < n_ch:
            start(c + 2)


def kernel(proposals, predict_scores, predict_deltas, predict_masks):
    f32 = jnp.float32
    b, n = predict_scores.shape
    pad = _NP - n
    scores_p = jnp.pad(predict_scores, ((0, 0), (0, pad)),
                       constant_values=-1.0)
    props_p = jnp.pad(proposals, ((0, 0), (0, pad), (0, 0)))
    deltas_p = jnp.pad(predict_deltas, ((0, 0), (0, pad), (0, 0)))
    s_row = scores_p[:, None, :]
    s_col = scores_p[:, :, None]
    props_t = jnp.swapaxes(props_p, 1, 2)
    deltas_t = jnp.swapaxes(deltas_p, 1, 2)

    boxes128, gidx128 = pl.pallas_call(
        _detect_body,
        grid=(b,),
        in_specs=[
            pl.BlockSpec((1, 1, _NP), lambda i: (i, 0, 0)),
            pl.BlockSpec((1, _NP, 1), lambda i: (i, 0, 0)),
            pl.BlockSpec((1, _NP, 6), lambda i: (i, 0, 0)),
            pl.BlockSpec((1, _NP, 6), lambda i: (i, 0, 0)),
            pl.BlockSpec((1, 6, _NP), lambda i: (i, 0, 0)),
            pl.BlockSpec((1, 6, _NP), lambda i: (i, 0, 0)),
        ],
        out_specs=[
            pl.BlockSpec((1, 128, 6), lambda i: (i, 0, 0)),
            pl.BlockSpec((1, 128, 1), lambda i: (i, 0, 0)),
        ],
        out_shape=[
            jax.ShapeDtypeStruct((b, 128, 6), f32),
            jax.ShapeDtypeStruct((b, 128, 1), f32),
        ],
        scratch_shapes=[pltpu.VMEM((_NP, _NP), f32)],
    )(s_row, s_col, props_p, deltas_p, props_t, deltas_t)

    sel_boxes = boxes128[:, :_K_OUT, :].reshape(b * _K_OUT, 6)
    gidx = jnp.round(gidx128[:, :_K_OUT, 0]).astype(jnp.int32)
    gidx_flat = (gidx + jnp.arange(b, dtype=jnp.int32)[:, None] * n).reshape(-1)

    masks2d = predict_masks.reshape(b * n, _MASK_D)
    table = jnp.pad(masks2d, ((0, 0), (0, _MASK_DP - _MASK_D)))
    table = table.reshape(b * n * _SUB_K, _SUB_D)
    idx_sub = (gidx_flat[:, None] * _SUB_K
               + jnp.arange(_SUB_K, dtype=jnp.int32)[None, :]).reshape(-1)
    idx_sub = jnp.pad(idx_sub, (0, _SUB_TOT - b * _K_OUT * _SUB_K))

    gather = pl.kernel(
        _sc_gather_body,
        out_type=jax.ShapeDtypeStruct((_SUB_TOT, _SUB_D), f32),
        mesh=plsc.VectorSubcoreMesh(core_axis_name="c", subcore_axis_name="s"),
        scratch_types=[
            pltpu.VMEM((_SUB_W,), jnp.int32),
            pltpu.VMEM((_SUB_CH, _SUB_D), f32),
            pltpu.VMEM((_SUB_CH, _SUB_D), f32),
            pltpu.SemaphoreType.DMA,
            pltpu.SemaphoreType.DMA,
        ],
    )
    gathered = gather(table, idx_sub)
    sel_masks = gathered[:b * _K_OUT * _SUB_K].reshape(b * _K_OUT, _MASK_DP)
    sel_masks = sel_masks[:, :_MASK_D].reshape(b * _K_OUT, 1, 28, 28, 28)

    batch_ids = jnp.repeat(jnp.arange(b, dtype=f32), _K_OUT)
    return sel_boxes, sel_masks, batch_ids


# speedup vs baseline: 1.8772x; 1.8772x over previous
import functools

import jax
import jax.numpy as jnp
from jax import lax
from jax.experimental import pallas as pl
from jax.experimental.pallas import tpu as pltpu
from jax.experimental.pallas import tpu_sc as plsc

_N = 1000
_NP = 1024
_K_OUT = 100
_THRESH = 0.3
_MASK_D = 28 * 28 * 28


def _detect_body(s_row_ref, s_col_ref, props_ref, deltas_ref, props_t_ref,
                 deltas_t_ref, boxes_out_ref, gidx_out_ref, iou_ref):
    f32 = jnp.float32
    s_row = s_row_ref[0]
    s_col = s_col_ref[0]
    lane = lax.broadcasted_iota(jnp.int32, (_NP, _NP), 1)
    sub = lax.broadcasted_iota(jnp.int32, (_NP, _NP), 0)

    cmp = jnp.where((s_row > s_col) | ((s_row == s_col) & (lane < sub)),
                    f32(1.0), f32(0.0))
    rank_col = jnp.sum(cmp, axis=1, keepdims=True)
    cmp_t = jnp.where((s_col > s_row) | ((s_col == s_row) & (sub < lane)),
                      f32(1.0), f32(0.0))
    rank_row = jnp.sum(cmp_t, axis=0, keepdims=True)

    sub_f = sub.astype(f32)
    lane_f = lane.astype(f32)
    perm = jnp.where(rank_row == sub_f, f32(1.0), f32(0.0))
    perm_t = jnp.where(rank_col == lane_f, f32(1.0), f32(0.0))

    dot = functools.partial(jnp.dot, preferred_element_type=f32,
                            precision=lax.Precision.HIGHEST)

    def regress(p, d, axis):
        def g(a, i):
            if axis == 1:
                return lax.slice_in_dim(a, i, i + 1, axis=1)
            return lax.slice_in_dim(a, i, i + 1, axis=0)
        y1, x1, z1, y2, x2, z2 = (g(p, i) for i in range(6))
        dy, dx, dz, dh, dw, dd = (g(d, i) for i in range(6))
        h = y2 - y1
        w = x2 - x1
        dep = z2 - z1
        cy = y1 + 0.5 * h + dy * h
        cx = x1 + 0.5 * w + dx * w
        cz = z1 + 0.5 * dep + dz * dep
        h = h * jnp.exp(dh)
        w = w * jnp.exp(dw)
        dep = dep * jnp.exp(dd)
        parts = [cy - 0.5 * h, cx - 0.5 * w, cz - 0.5 * dep,
                 cy + 0.5 * h, cx + 0.5 * w, cz + 0.5 * dep]
        return jnp.concatenate(parts, axis=1 if axis == 1 else 0)

    bb_col = regress(props_ref[0], deltas_ref[0], axis=1)
    bb_row = regress(props_t_ref[0], deltas_t_ref[0], axis=0)
    sbox_c = dot(perm, bb_col)
    sbox_r = dot(bb_row, perm_t)

    def col(i):
        return lax.slice_in_dim(sbox_c, i, i + 1, axis=1)

    def row(i):
        return lax.slice_in_dim(sbox_r, i, i + 1, axis=0)

    y1c, x1c, z1c, y2c, x2c, z2c = (col(i) for i in range(6))
    y1r, x1r, z1r, y2r, x2r, z2r = (row(i) for i in range(6))
    zero = f32(0.0)
    inter = (jnp.maximum(jnp.minimum(y2c, y2r) - jnp.maximum(y1c, y1r), zero)
             * jnp.maximum(jnp.minimum(x2c, x2r) - jnp.maximum(x1c, x1r), zero)
             * jnp.maximum(jnp.minimum(z2c, z2r) - jnp.maximum(z1c, z1r), zero))
    vol_c = (y2c - y1c) * (x2c - x1c) * (z2c - z1c)
    vol_r = (y2r - y1r) * (x2r - x1r) * (z2r - z1r)
    iou_ref[...] = inter / (vol_c + vol_r - inter + f32(1e-8))

    lane_row = lax.broadcasted_iota(jnp.int32, (1, _NP), 1)
    lane_row_f = lane_row.astype(f32)

    def nms_step(i, keep):
        iou_row = iou_ref[pl.ds(i, 1), :]
        keep_i = jnp.sum(jnp.where(lane_row == i, keep, zero))
        suppress = (iou_row > _THRESH) & (lane_row > i) & (keep_i > zero)
        return jnp.where(suppress, zero, keep)

    keep = lax.fori_loop(0, _N, nms_step, jnp.ones((1, _NP), f32))

    validm = lane_row < _N
    keptf = jnp.where(validm, keep, zero)
    suppf = jnp.where(validm, 1.0 - keep, zero)
    tri = jnp.where(sub <= lane, f32(1.0), f32(0.0))
    csum_kept = dot(keptf, tri)
    csum_supp = dot(suppf, tri)
    n_kept = jnp.sum(keptf)
    slot = jnp.where(keptf > zero, csum_kept - 1.0,
                     jnp.where(suppf > zero, n_kept + csum_supp - 1.0,
                               f32(1e9)))

    sel_sub = lax.broadcasted_iota(jnp.int32, (128, _NP), 0).astype(f32)
    sel = jnp.where(slot == sel_sub, f32(1.0), f32(0.0))
    boxes_out_ref[0] = dot(sel, sbox_c)
    sorted_gidx = dot(perm, sub_f[:, 0:1])
    gidx_out_ref[0] = dot(sel, sorted_gidx)


_SC_NC = 2
_SC_NS = 16
_SC_NW = _SC_NC * _SC_NS
_MASK_DP = 22016
_SUB_K = 4
_SUB_D = _MASK_DP // _SUB_K
_SUB_TOT = 1024
_SUB_W = _SUB_TOT // _SC_NW
_SUB_CH = 8


def _gather_body(idx_ref, *refs):
    out_ref = refs[-1]
    for k, in_ref in enumerate(refs[:-1]):
        out_ref[k] = in_ref[0]


def kernel(proposals, predict_scores, predict_deltas, predict_masks):
    f32 = jnp.float32
    b, n = predict_scores.shape
    pad = _NP - n
    scores_p = jnp.pad(predict_scores, ((0, 0), (0, pad)),
                       constant_values=-1.0)
    props_p = jnp.pad(proposals, ((0, 0), (0, pad), (0, 0)))
    deltas_p = jnp.pad(predict_deltas, ((0, 0), (0, pad), (0, 0)))
    s_row = scores_p[:, None, :]
    s_col = scores_p[:, :, None]
    props_t = jnp.swapaxes(props_p, 1, 2)
    deltas_t = jnp.swapaxes(deltas_p, 1, 2)

    boxes128, gidx128 = pl.pallas_call(
        _detect_body,
        grid=(b,),
        in_specs=[
            pl.BlockSpec((1, 1, _NP), lambda i: (i, 0, 0)),
            pl.BlockSpec((1, _NP, 1), lambda i: (i, 0, 0)),
            pl.BlockSpec((1, _NP, 6), lambda i: (i, 0, 0)),
            pl.BlockSpec((1, _NP, 6), lambda i: (i, 0, 0)),
            pl.BlockSpec((1, 6, _NP), lambda i: (i, 0, 0)),
            pl.BlockSpec((1, 6, _NP), lambda i: (i, 0, 0)),
        ],
        out_specs=[
            pl.BlockSpec((1, 128, 6), lambda i: (i, 0, 0)),
            pl.BlockSpec((1, 128, 1), lambda i: (i, 0, 0)),
        ],
        out_shape=[
            jax.ShapeDtypeStruct((b, 128, 6), f32),
            jax.ShapeDtypeStruct((b, 128, 1), f32),
        ],
        scratch_shapes=[pltpu.VMEM((_NP, _NP), f32)],
    )(s_row, s_col, props_p, deltas_p, props_t, deltas_t)

    sel_boxes = boxes128[:, :_K_OUT, :].reshape(b * _K_OUT, 6)
    gidx = jnp.round(gidx128[:, :_K_OUT, 0]).astype(jnp.int32)
    gidx_flat = (gidx + jnp.arange(b, dtype=jnp.int32)[:, None] * n).reshape(-1)

    masks3d = predict_masks.reshape(b * n, 8, _MASK_D // 8)
    group = 20
    in_specs = [
        pl.BlockSpec((1, 8, _MASK_D // 8),
                     (lambda i, idx_ref, k=k: (idx_ref[group * i + k], 0, 0)))
        for k in range(group)
    ]
    sel_masks = pl.pallas_call(
        _gather_body,
        grid_spec=pltpu.PrefetchScalarGridSpec(
            num_scalar_prefetch=1,
            grid=(b * _K_OUT // group,),
            in_specs=in_specs,
            out_specs=pl.BlockSpec((group, 8, _MASK_D // 8),
                                   lambda i, idx_ref: (i, 0, 0)),
        ),
        out_shape=jax.ShapeDtypeStruct((b * _K_OUT, 8, _MASK_D // 8), f32),
    )(gidx_flat, *([masks3d] * group))
    sel_masks = sel_masks.reshape(b * _K_OUT, 1, 28, 28, 28)

    batch_ids = jnp.repeat(jnp.arange(b, dtype=f32), _K_OUT)
    return sel_boxes, sel_masks, batch_ids


# speedup vs baseline: 1.9153x; 1.0203x over previous
import functools

import jax
import jax.numpy as jnp
from jax import lax
from jax.experimental import pallas as pl
from jax.experimental.pallas import tpu as pltpu
from jax.experimental.pallas import tpu_sc as plsc

_N = 1000
_NP = 1024
_K_OUT = 100
_THRESH = 0.3
_MASK_D = 28 * 28 * 28


def _detect_body(s_row_ref, s_col_ref, props_ref, deltas_ref, props_t_ref,
                 deltas_t_ref, boxes_out_ref, gidx_out_ref, iou_ref):
    f32 = jnp.float32
    s_row = s_row_ref[0]
    s_col = s_col_ref[0]
    lane = lax.broadcasted_iota(jnp.int32, (_NP, _NP), 1)
    sub = lax.broadcasted_iota(jnp.int32, (_NP, _NP), 0)

    cmp = jnp.where((s_row > s_col) | ((s_row == s_col) & (lane < sub)),
                    f32(1.0), f32(0.0))
    rank_col = jnp.sum(cmp, axis=1, keepdims=True)
    cmp_t = jnp.where((s_col > s_row) | ((s_col == s_row) & (sub < lane)),
                      f32(1.0), f32(0.0))
    rank_row = jnp.sum(cmp_t, axis=0, keepdims=True)

    sub_f = sub.astype(f32)
    lane_f = lane.astype(f32)
    perm = jnp.where(rank_row == sub_f, f32(1.0), f32(0.0))
    perm_t = jnp.where(rank_col == lane_f, f32(1.0), f32(0.0))

    dot = functools.partial(jnp.dot, preferred_element_type=f32,
                            precision=lax.Precision.HIGHEST)

    def regress(p, d, axis):
        def g(a, i):
            if axis == 1:
                return lax.slice_in_dim(a, i, i + 1, axis=1)
            return lax.slice_in_dim(a, i, i + 1, axis=0)
        y1, x1, z1, y2, x2, z2 = (g(p, i) for i in range(6))
        dy, dx, dz, dh, dw, dd = (g(d, i) for i in range(6))
        h = y2 - y1
        w = x2 - x1
        dep = z2 - z1
        cy = y1 + 0.5 * h + dy * h
        cx = x1 + 0.5 * w + dx * w
        cz = z1 + 0.5 * dep + dz * dep
        h = h * jnp.exp(dh)
        w = w * jnp.exp(dw)
        dep = dep * jnp.exp(dd)
        parts = [cy - 0.5 * h, cx - 0.5 * w, cz - 0.5 * dep,
                 cy + 0.5 * h, cx + 0.5 * w, cz + 0.5 * dep]
        return jnp.concatenate(parts, axis=1 if axis == 1 else 0)

    bb_col = regress(props_ref[0], deltas_ref[0], axis=1)
    bb_row = regress(props_t_ref[0], deltas_t_ref[0], axis=0)
    sbox_c = dot(perm, bb_col)
    sbox_r = dot(bb_row, perm_t)

    def col(i):
        return lax.slice_in_dim(sbox_c, i, i + 1, axis=1)

    def row(i):
        return lax.slice_in_dim(sbox_r, i, i + 1, axis=0)

    y1c, x1c, z1c, y2c, x2c, z2c = (col(i) for i in range(6))
    y1r, x1r, z1r, y2r, x2r, z2r = (row(i) for i in range(6))
    zero = f32(0.0)
    inter = (jnp.maximum(jnp.minimum(y2c, y2r) - jnp.maximum(y1c, y1r), zero)
             * jnp.maximum(jnp.minimum(x2c, x2r) - jnp.maximum(x1c, x1r), zero)
             * jnp.maximum(jnp.minimum(z2c, z2r) - jnp.maximum(z1c, z1r), zero))
    vol_c = (y2c - y1c) * (x2c - x1c) * (z2c - z1c)
    vol_r = (y2r - y1r) * (x2r - x1r) * (z2r - z1r)
    iou_ref[...] = inter / (vol_c + vol_r - inter + f32(1e-8))

    lane_row = lax.broadcasted_iota(jnp.int32, (1, _NP), 1)

    blk = 128
    th = f32(_THRESH)
    one = f32(1.0)
    lane_b = lax.broadcasted_iota(jnp.int32, (1, blk), 1)
    keep_parts = []
    for t in range(_NP // blk):
        c0 = t * blk
        tsub = jnp.where(iou_ref[c0:c0 + blk, c0:c0 + blk] > th, one, zero)
        if t == 0:
            keep_sub = jnp.ones((1, blk), f32)
        else:
            kept_prev = jnp.concatenate(
                keep_parts + [jnp.zeros((1, _NP - c0), f32)], axis=1)
            tcol = jnp.where(iou_ref[:, c0:c0 + blk] > th, one, zero)
            n_sup = dot(kept_prev, tcol)
            keep_sub = jnp.where(n_sup > zero, zero, one)
        for k in range(blk):
            row_k = tsub[k:k + 1, :]
            alive = jnp.sum(jnp.where(lane_b == k, keep_sub, zero))
            suppress = (row_k > zero) & (lane_b > k) & (alive > zero)
            keep_sub = jnp.where(suppress, zero, keep_sub)
        keep_parts.append(keep_sub)
    keep = jnp.concatenate(keep_parts, axis=1)

    validm = lane_row < _N
    keptf = jnp.where(validm, keep, zero)
    suppf = jnp.where(validm, 1.0 - keep, zero)
    tri = jnp.where(sub <= lane, f32(1.0), f32(0.0))
    csum_kept = dot(keptf, tri)
    csum_supp = dot(suppf, tri)
    n_kept = jnp.sum(keptf)
    slot = jnp.where(keptf > zero, csum_kept - 1.0,
                     jnp.where(suppf > zero, n_kept + csum_supp - 1.0,
                               f32(1e9)))

    sel_sub = lax.broadcasted_iota(jnp.int32, (128, _NP), 0).astype(f32)
    sel = jnp.where(slot == sel_sub, f32(1.0), f32(0.0))
    boxes_out_ref[0] = dot(sel, sbox_c)
    sorted_gidx = dot(perm, sub_f[:, 0:1])
    gidx_out_ref[0] = dot(sel, sorted_gidx)


_SC_NC = 2
_SC_NS = 16
_SC_NW = _SC_NC * _SC_NS
_MASK_DP = 22016
_SUB_K = 4
_SUB_D = _MASK_DP // _SUB_K
_SUB_TOT = 1024
_SUB_W = _SUB_TOT // _SC_NW
_SUB_CH = 8


def _gather_body(idx_ref, *refs):
    out_ref = refs[-1]
    for k, in_ref in enumerate(refs[:-1]):
        out_ref[k] = in_ref[0]


def kernel(proposals, predict_scores, predict_deltas, predict_masks):
    f32 = jnp.float32
    b, n = predict_scores.shape
    pad = _NP - n
    scores_p = jnp.pad(predict_scores, ((0, 0), (0, pad)),
                       constant_values=-1.0)
    props_p = jnp.pad(proposals, ((0, 0), (0, pad), (0, 0)))
    deltas_p = jnp.pad(predict_deltas, ((0, 0), (0, pad), (0, 0)))
    s_row = scores_p[:, None, :]
    s_col = scores_p[:, :, None]
    props_t = jnp.swapaxes(props_p, 1, 2)
    deltas_t = jnp.swapaxes(deltas_p, 1, 2)

    boxes128, gidx128 = pl.pallas_call(
        _detect_body,
        grid=(b,),
        in_specs=[
            pl.BlockSpec((1, 1, _NP), lambda i: (i, 0, 0)),
            pl.BlockSpec((1, _NP, 1), lambda i: (i, 0, 0)),
            pl.BlockSpec((1, _NP, 6), lambda i: (i, 0, 0)),
            pl.BlockSpec((1, _NP, 6), lambda i: (i, 0, 0)),
            pl.BlockSpec((1, 6, _NP), lambda i: (i, 0, 0)),
            pl.BlockSpec((1, 6, _NP), lambda i: (i, 0, 0)),
        ],
        out_specs=[
            pl.BlockSpec((1, 128, 6), lambda i: (i, 0, 0)),
            pl.BlockSpec((1, 128, 1), lambda i: (i, 0, 0)),
        ],
        out_shape=[
            jax.ShapeDtypeStruct((b, 128, 6), f32),
            jax.ShapeDtypeStruct((b, 128, 1), f32),
        ],
        scratch_shapes=[pltpu.VMEM((_NP, _NP), f32)],
    )(s_row, s_col, props_p, deltas_p, props_t, deltas_t)

    sel_boxes = boxes128[:, :_K_OUT, :].reshape(b * _K_OUT, 6)
    gidx = jnp.round(gidx128[:, :_K_OUT, 0]).astype(jnp.int32)
    gidx_flat = (gidx + jnp.arange(b, dtype=jnp.int32)[:, None] * n).reshape(-1)

    masks3d = predict_masks.reshape(b * n, 8, _MASK_D // 8)
    group = 20
    in_specs = [
        pl.BlockSpec((1, 8, _MASK_D // 8),
                     (lambda i, idx_ref, k=k: (idx_ref[group * i + k], 0, 0)))
        for k in range(group)
    ]
    sel_masks = pl.pallas_call(
        _gather_body,
        grid_spec=pltpu.PrefetchScalarGridSpec(
            num_scalar_prefetch=1,
            grid=(b * _K_OUT // group,),
            in_specs=in_specs,
            out_specs=pl.BlockSpec((group, 8, _MASK_D // 8),
                                   lambda i, idx_ref: (i, 0, 0)),
        ),
        out_shape=jax.ShapeDtypeStruct((b * _K_OUT, 8, _MASK_D // 8), f32),
    )(gidx_flat, *([masks3d] * group))
    sel_masks = sel_masks.reshape(b * _K_OUT, 1, 28, 28, 28)

    batch_ids = jnp.repeat(jnp.arange(b, dtype=f32), _K_OUT)
    return sel_boxes, sel_masks, batch_ids


# speedup vs baseline: 2.3468x; 1.2253x over previous
import functools

import jax
import jax.numpy as jnp
from jax import lax
from jax.experimental import pallas as pl
from jax.experimental.pallas import tpu as pltpu
from jax.experimental.pallas import tpu_sc as plsc

_N = 1000
_NP = 1024
_K_OUT = 100
_THRESH = 0.3
_MASK_D = 28 * 28 * 28


def _detect_body(s_row_ref, s_col_ref, props_ref, deltas_ref, props_t_ref,
                 deltas_t_ref, boxes_out_ref, gidx_out_ref, iou_ref):
    f32 = jnp.float32
    s_row = s_row_ref[0]
    s_col = s_col_ref[0]
    lane = lax.broadcasted_iota(jnp.int32, (_NP, _NP), 1)
    sub = lax.broadcasted_iota(jnp.int32, (_NP, _NP), 0)

    cmp = jnp.where((s_row > s_col) | ((s_row == s_col) & (lane < sub)),
                    f32(1.0), f32(0.0))
    rank_col = jnp.sum(cmp, axis=1, keepdims=True)
    cmp_t = jnp.where((s_col > s_row) | ((s_col == s_row) & (sub < lane)),
                      f32(1.0), f32(0.0))
    rank_row = jnp.sum(cmp_t, axis=0, keepdims=True)

    sub_f = sub.astype(f32)
    lane_f = lane.astype(f32)
    perm = jnp.where(rank_row == sub_f, f32(1.0), f32(0.0))
    perm_t = jnp.where(rank_col == lane_f, f32(1.0), f32(0.0))

    dot = functools.partial(jnp.dot, preferred_element_type=f32,
                            precision=lax.Precision.HIGHEST)

    def regress(p, d, axis):
        def g(a, i):
            if axis == 1:
                return lax.slice_in_dim(a, i, i + 1, axis=1)
            return lax.slice_in_dim(a, i, i + 1, axis=0)
        y1, x1, z1, y2, x2, z2 = (g(p, i) for i in range(6))
        dy, dx, dz, dh, dw, dd = (g(d, i) for i in range(6))
        h = y2 - y1
        w = x2 - x1
        dep = z2 - z1
        cy = y1 + 0.5 * h + dy * h
        cx = x1 + 0.5 * w + dx * w
        cz = z1 + 0.5 * dep + dz * dep
        h = h * jnp.exp(dh)
        w = w * jnp.exp(dw)
        dep = dep * jnp.exp(dd)
        parts = [cy - 0.5 * h, cx - 0.5 * w, cz - 0.5 * dep,
                 cy + 0.5 * h, cx + 0.5 * w, cz + 0.5 * dep]
        return jnp.concatenate(parts, axis=1 if axis == 1 else 0)

    bb_col = regress(props_ref[0], deltas_ref[0], axis=1)
    bb_row = regress(props_t_ref[0], deltas_t_ref[0], axis=0)
    sbox_c = dot(perm, bb_col)
    sbox_r = dot(bb_row, perm_t)

    def col(i):
        return lax.slice_in_dim(sbox_c, i, i + 1, axis=1)

    def row(i):
        return lax.slice_in_dim(sbox_r, i, i + 1, axis=0)

    y1c, x1c, z1c, y2c, x2c, z2c = (col(i) for i in range(6))
    y1r, x1r, z1r, y2r, x2r, z2r = (row(i) for i in range(6))
    zero = f32(0.0)
    inter = (jnp.maximum(jnp.minimum(y2c, y2r) - jnp.maximum(y1c, y1r), zero)
             * jnp.maximum(jnp.minimum(x2c, x2r) - jnp.maximum(x1c, x1r), zero)
             * jnp.maximum(jnp.minimum(z2c, z2r) - jnp.maximum(z1c, z1r), zero))
    vol_c = (y2c - y1c) * (x2c - x1c) * (z2c - z1c)
    vol_r = (y2r - y1r) * (x2r - x1r) * (z2r - z1r)
    iou = inter / (vol_c + vol_r - inter + f32(1e-8))

    lane_row = lax.broadcasted_iota(jnp.int32, (1, _NP), 1)
    one = f32(1.0)

    iou_ref[...] = jnp.where((iou > _THRESH) & (sub < lane), one, zero)

    def peel_cond(carry):
        und, _ = carry
        return jnp.sum(und) > zero

    def peel_body(carry):
        und, kept = carry
        blockers = jnp.maximum(und, kept)
        n_pred = dot(blockers, iou_ref[...])
        newkeep = jnp.where((und > zero) & (n_pred == zero), one, zero)
        kept = jnp.maximum(kept, newkeep)
        n_sup = dot(kept, iou_ref[...])
        und = jnp.where((newkeep > zero) | (n_sup > zero), zero, und)
        return und, kept

    _, keep = lax.while_loop(
        peel_cond, peel_body,
        (jnp.ones((1, _NP), f32), jnp.zeros((1, _NP), f32)))

    validm = lane_row < _N
    keptf = jnp.where(validm, keep, zero)
    suppf = jnp.where(validm, 1.0 - keep, zero)
    tri = jnp.where(sub <= lane, f32(1.0), f32(0.0))
    csum_kept = dot(keptf, tri)
    csum_supp = dot(suppf, tri)
    n_kept = jnp.sum(keptf)
    slot = jnp.where(keptf > zero, csum_kept - 1.0,
                     jnp.where(suppf > zero, n_kept + csum_supp - 1.0,
                               f32(1e9)))

    sel_sub = lax.broadcasted_iota(jnp.int32, (128, _NP), 0).astype(f32)
    sel = jnp.where(slot == sel_sub, f32(1.0), f32(0.0))
    boxes_out_ref[0] = dot(sel, sbox_c)
    sorted_gidx = dot(perm, sub_f[:, 0:1])
    gidx_out_ref[0] = dot(sel, sorted_gidx)


_SC_NC = 2
_SC_NS = 16
_SC_NW = _SC_NC * _SC_NS
_MASK_DP = 22016
_SUB_K = 4
_SUB_D = _MASK_DP // _SUB_K
_SUB_TOT = 1024
_SUB_W = _SUB_TOT // _SC_NW
_SUB_CH = 8


def _gather_body(idx_ref, *refs):
    out_ref = refs[-1]
    for k, in_ref in enumerate(refs[:-1]):
        out_ref[k] = in_ref[0]


def kernel(proposals, predict_scores, predict_deltas, predict_masks):
    f32 = jnp.float32
    b, n = predict_scores.shape
    pad = _NP - n
    scores_p = jnp.pad(predict_scores, ((0, 0), (0, pad)),
                       constant_values=-1.0)
    props_p = jnp.pad(proposals, ((0, 0), (0, pad), (0, 0)))
    deltas_p = jnp.pad(predict_deltas, ((0, 0), (0, pad), (0, 0)))
    s_row = scores_p[:, None, :]
    s_col = scores_p[:, :, None]
    props_t = jnp.swapaxes(props_p, 1, 2)
    deltas_t = jnp.swapaxes(deltas_p, 1, 2)

    boxes128, gidx128 = pl.pallas_call(
        _detect_body,
        grid=(b,),
        in_specs=[
            pl.BlockSpec((1, 1, _NP), lambda i: (i, 0, 0)),
            pl.BlockSpec((1, _NP, 1), lambda i: (i, 0, 0)),
            pl.BlockSpec((1, _NP, 6), lambda i: (i, 0, 0)),
            pl.BlockSpec((1, _NP, 6), lambda i: (i, 0, 0)),
            pl.BlockSpec((1, 6, _NP), lambda i: (i, 0, 0)),
            pl.BlockSpec((1, 6, _NP), lambda i: (i, 0, 0)),
        ],
        out_specs=[
            pl.BlockSpec((1, 128, 6), lambda i: (i, 0, 0)),
            pl.BlockSpec((1, 128, 1), lambda i: (i, 0, 0)),
        ],
        out_shape=[
            jax.ShapeDtypeStruct((b, 128, 6), f32),
            jax.ShapeDtypeStruct((b, 128, 1), f32),
        ],
        scratch_shapes=[pltpu.VMEM((_NP, _NP), f32)],
    )(s_row, s_col, props_p, deltas_p, props_t, deltas_t)

    sel_boxes = boxes128[:, :_K_OUT, :].reshape(b * _K_OUT, 6)
    gidx = jnp.round(gidx128[:, :_K_OUT, 0]).astype(jnp.int32)
    gidx_flat = (gidx + jnp.arange(b, dtype=jnp.int32)[:, None] * n).reshape(-1)

    masks3d = predict_masks.reshape(b * n, 8, _MASK_D // 8)
    group = 20
    in_specs = [
        pl.BlockSpec((1, 8, _MASK_D // 8),
                     (lambda i, idx_ref, k=k: (idx_ref[group * i + k], 0, 0)))
        for k in range(group)
    ]
    sel_masks = pl.pallas_call(
        _gather_body,
        grid_spec=pltpu.PrefetchScalarGridSpec(
            num_scalar_prefetch=1,
            grid=(b * _K_OUT // group,),
            in_specs=in_specs,
            out_specs=pl.BlockSpec((group, 8, _MASK_D // 8),
                                   lambda i, idx_ref: (i, 0, 0)),
        ),
        out_shape=jax.ShapeDtypeStruct((b * _K_OUT, 8, _MASK_D // 8), f32),
    )(gidx_flat, *([masks3d] * group))
    sel_masks = sel_masks.reshape(b * _K_OUT, 1, 28, 28, 28)

    batch_ids = jnp.repeat(jnp.arange(b, dtype=f32), _K_OUT)
    return sel_boxes, sel_masks, batch_ids


# speedup vs baseline: 2.3505x; 1.0016x over previous
import functools

import jax
import jax.numpy as jnp
from jax import lax
from jax.experimental import pallas as pl
from jax.experimental.pallas import tpu as pltpu

_N = 1000
_NP = 1024
_K_OUT = 100
_THRESH = 0.3
_MASK_D = 28 * 28 * 28


def _detect_body(s_row_ref, s_col_ref, props_ref, deltas_ref, props_t_ref,
                 deltas_t_ref, boxes_out_ref, gidx_out_ref, iou_ref):
    f32 = jnp.float32
    s_row = s_row_ref[0]
    s_col = s_col_ref[0]
    lane = lax.broadcasted_iota(jnp.int32, (_NP, _NP), 1)
    sub = lax.broadcasted_iota(jnp.int32, (_NP, _NP), 0)

    cmp = jnp.where((s_row > s_col) | ((s_row == s_col) & (lane < sub)),
                    f32(1.0), f32(0.0))
    rank_col = jnp.sum(cmp, axis=1, keepdims=True)
    cmp_t = jnp.where((s_col > s_row) | ((s_col == s_row) & (sub < lane)),
                      f32(1.0), f32(0.0))
    rank_row = jnp.sum(cmp_t, axis=0, keepdims=True)

    sub_f = sub.astype(f32)
    lane_f = lane.astype(f32)
    perm = jnp.where(rank_row == sub_f, f32(1.0), f32(0.0))
    perm_t = jnp.where(rank_col == lane_f, f32(1.0), f32(0.0))

    dot = functools.partial(jnp.dot, preferred_element_type=f32,
                            precision=lax.Precision.HIGHEST)

    def regress(p, d, axis):
        def g(a, i):
            if axis == 1:
                return lax.slice_in_dim(a, i, i + 1, axis=1)
            return lax.slice_in_dim(a, i, i + 1, axis=0)
        y1, x1, z1, y2, x2, z2 = (g(p, i) for i in range(6))
        dy, dx, dz, dh, dw, dd = (g(d, i) for i in range(6))
        h = y2 - y1
        w = x2 - x1
        dep = z2 - z1
        cy = y1 + 0.5 * h + dy * h
        cx = x1 + 0.5 * w + dx * w
        cz = z1 + 0.5 * dep + dz * dep
        h = h * jnp.exp(dh)
        w = w * jnp.exp(dw)
        dep = dep * jnp.exp(dd)
        parts = [cy - 0.5 * h, cx - 0.5 * w, cz - 0.5 * dep,
                 cy + 0.5 * h, cx + 0.5 * w, cz + 0.5 * dep]
        return jnp.concatenate(parts, axis=1 if axis == 1 else 0)

    bb_col = regress(props_ref[0], deltas_ref[0], axis=1)
    bb_row = regress(props_t_ref[0], deltas_t_ref[0], axis=0)
    sbox_c = dot(perm, bb_col)
    sbox_r = dot(bb_row, perm_t)

    def col(i):
        return lax.slice_in_dim(sbox_c, i, i + 1, axis=1)

    def row(i):
        return lax.slice_in_dim(sbox_r, i, i + 1, axis=0)

    y1c, x1c, z1c, y2c, x2c, z2c = (col(i) for i in range(6))
    y1r, x1r, z1r, y2r, x2r, z2r = (row(i) for i in range(6))
    zero = f32(0.0)
    inter = (jnp.maximum(jnp.minimum(y2c, y2r) - jnp.maximum(y1c, y1r), zero)
             * jnp.maximum(jnp.minimum(x2c, x2r) - jnp.maximum(x1c, x1r), zero)
             * jnp.maximum(jnp.minimum(z2c, z2r) - jnp.maximum(z1c, z1r), zero))
    vol_c = (y2c - y1c) * (x2c - x1c) * (z2c - z1c)
    vol_r = (y2r - y1r) * (x2r - x1r) * (z2r - z1r)
    iou = inter / (vol_c + vol_r - inter + f32(1e-8))

    lane_row = lax.broadcasted_iota(jnp.int32, (1, _NP), 1)
    one = f32(1.0)

    iou_ref[...] = jnp.where((iou > _THRESH) & (sub < lane), one, zero)

    def peel_cond(carry):
        und, _ = carry
        return jnp.sum(und) > zero

    def peel_body(carry):
        und, kept = carry
        blockers = jnp.maximum(und, kept)
        n_pred = dot(blockers, iou_ref[...])
        newkeep = jnp.where((und > zero) & (n_pred == zero), one, zero)
        kept = jnp.maximum(kept, newkeep)
        n_sup = dot(kept, iou_ref[...])
        und = jnp.where((newkeep > zero) | (n_sup > zero), zero, und)
        return und, kept

    _, keep = lax.while_loop(
        peel_cond, peel_body,
        (jnp.ones((1, _NP), f32), jnp.zeros((1, _NP), f32)))

    validm = lane_row < _N
    keptf = jnp.where(validm, keep, zero)
    suppf = jnp.where(validm, 1.0 - keep, zero)
    tri = jnp.where(sub <= lane, f32(1.0), f32(0.0))
    csum_kept = dot(keptf, tri)
    csum_supp = dot(suppf, tri)
    n_kept = jnp.sum(keptf)
    slot = jnp.where(keptf > zero, csum_kept - 1.0,
                     jnp.where(suppf > zero, n_kept + csum_supp - 1.0,
                               f32(1e9)))

    sel_sub = lax.broadcasted_iota(jnp.int32, (128, _NP), 0).astype(f32)
    sel = jnp.where(slot == sel_sub, f32(1.0), f32(0.0))
    boxes_out_ref[0] = dot(sel, sbox_c)
    sorted_gidx = dot(perm, sub_f[:, 0:1])
    gidx_out_ref[0] = dot(sel, sorted_gidx)


def _gather_body(idx_ref, *refs):
    out_ref = refs[-1]
    for k, in_ref in enumerate(refs[:-1]):
        out_ref[k] = in_ref[0]


def kernel(proposals, predict_scores, predict_deltas, predict_masks):
    f32 = jnp.float32
    b, n = predict_scores.shape
    pad = _NP - n
    scores_p = jnp.pad(predict_scores, ((0, 0), (0, pad)),
                       constant_values=-1.0)
    props_p = jnp.pad(proposals, ((0, 0), (0, pad), (0, 0)))
    deltas_p = jnp.pad(predict_deltas, ((0, 0), (0, pad), (0, 0)))
    s_row = scores_p[:, None, :]
    s_col = scores_p[:, :, None]
    props_t = jnp.swapaxes(props_p, 1, 2)
    deltas_t = jnp.swapaxes(deltas_p, 1, 2)

    boxes128, gidx128 = pl.pallas_call(
        _detect_body,
        grid=(b,),
        in_specs=[
            pl.BlockSpec((1, 1, _NP), lambda i: (i, 0, 0)),
            pl.BlockSpec((1, _NP, 1), lambda i: (i, 0, 0)),
            pl.BlockSpec((1, _NP, 6), lambda i: (i, 0, 0)),
            pl.BlockSpec((1, _NP, 6), lambda i: (i, 0, 0)),
            pl.BlockSpec((1, 6, _NP), lambda i: (i, 0, 0)),
            pl.BlockSpec((1, 6, _NP), lambda i: (i, 0, 0)),
        ],
        out_specs=[
            pl.BlockSpec((1, 128, 6), lambda i: (i, 0, 0)),
            pl.BlockSpec((1, 128, 1), lambda i: (i, 0, 0)),
        ],
        out_shape=[
            jax.ShapeDtypeStruct((b, 128, 6), f32),
            jax.ShapeDtypeStruct((b, 128, 1), f32),
        ],
        scratch_shapes=[pltpu.VMEM((_NP, _NP), f32)],
    )(s_row, s_col, props_p, deltas_p, props_t, deltas_t)

    sel_boxes = boxes128[:, :_K_OUT, :].reshape(b * _K_OUT, 6)
    gidx = jnp.round(gidx128[:, :_K_OUT, 0]).astype(jnp.int32)
    gidx_flat = (gidx + jnp.arange(b, dtype=jnp.int32)[:, None] * n).reshape(-1)

    masks3d = predict_masks.reshape(b * n, 8, _MASK_D // 8)
    group = 20
    in_specs = [
        pl.BlockSpec((1, 8, _MASK_D // 8),
                     (lambda i, idx_ref, k=k: (idx_ref[group * i + k], 0, 0)))
        for k in range(group)
    ]
    sel_masks = pl.pallas_call(
        _gather_body,
        grid_spec=pltpu.PrefetchScalarGridSpec(
            num_scalar_prefetch=1,
            grid=(b * _K_OUT // group,),
            in_specs=in_specs,
            out_specs=pl.BlockSpec((group, 8, _MASK_D // 8),
                                   lambda i, idx_ref: (i, 0, 0)),
        ),
        out_shape=jax.ShapeDtypeStruct((b * _K_OUT, 8, _MASK_D // 8), f32),
    )(gidx_flat, *([masks3d] * group))
    sel_masks = sel_masks.reshape(b * _K_OUT, 1, 28, 28, 28)

    batch_ids = jnp.repeat(jnp.arange(b, dtype=f32), _K_OUT)
    return sel_boxes, sel_masks, batch_ids
